# baseline (speedup 1.0000x reference)
import functools

import jax
import jax.numpy as jnp
from jax import lax
from jax.experimental import pallas as pl
from jax.experimental.pallas import tpu as pltpu
from jax.experimental.pallas import tpu_sc as plsc

B = 16384
TYPE_VOCAB = 1000
DOMAIN_DIM = 16
TYPE_DIM = 8
DAY_DIM = 4
HIDDEN = 128

NC = 2
NS = 16
LANES = 16
NW = NC * NS
DMA_CHUNK = 16


def _make_sc_gather(nb):
    bpw = nb // NW
    groups = bpw // LANES
    n_chunks = bpw // DMA_CHUNK

    def body(dom_t, typ_tab, day_tab, dom_id, typ_id, day_id,
             dm_out, t_out, d_out,
             didx_v, tiles_v, tid_v, did_v, ttab_v, dtab_v,
             dmblk_v, tblk_v, dblk_v, sem):
        wid = lax.axis_index("s") * NC + lax.axis_index("c")
        base = wid * bpw

        pltpu.sync_copy(dom_id.at[pl.ds(base, bpw)], didx_v)
        pltpu.sync_copy(typ_id.at[pl.ds(base, bpw)], tid_v)
        pltpu.sync_copy(day_id.at[pl.ds(base, bpw)], did_v)
        pltpu.sync_copy(typ_tab, ttab_v)
        pltpu.sync_copy(day_tab, dtab_v)

        lane = lax.iota(jnp.int32, LANES)

        def fire(ch):
            ids = didx_v[pl.ds(ch * DMA_CHUNK, DMA_CHUNK)]
            ks = lax.shift_right_logical(ids, 7)
            slot_base = (ch % 2) * DMA_CHUNK
            for j in range(DMA_CHUNK):
                off = pl.multiple_of(ks[j] * 128, 128)
                pltpu.async_copy(dom_t.at[:, pl.ds(off, 128)],
                                 tiles_v.at[slot_base + j], sem)

        def extract(ch):
            ids = didx_v[pl.ds(ch * DMA_CHUNK, DMA_CHUNK)]
            ls = ids & 127
            slot_base = (ch % 2) * DMA_CHUNK
            for j in range(DMA_CHUNK):
                pltpu.make_async_copy(dom_t.at[:, pl.ds(0, 128)],
                                      tiles_v.at[slot_base + j], sem).wait()
            for j in range(DMA_CHUNK):
                i = ch * DMA_CHUNK + j
                v = plsc.load_gather(
                    tiles_v, [jnp.broadcast_to(slot_base + j, (LANES,)),
                              lane, jnp.broadcast_to(ls[j], (LANES,))])
                dmblk_v[pl.ds(i * DOMAIN_DIM, DOMAIN_DIM)] = v

        def type_day_group(g):
            tids = tid_v[pl.ds(g * LANES, LANES)]
            tdst = (g * LANES + lane) * TYPE_DIM
            tsrc = tids * TYPE_DIM
            for j in range(TYPE_DIM):
                v = plsc.load_gather(ttab_v, [tsrc + j])
                plsc.store_scatter(tblk_v, [tdst + j], v)

            dids = did_v[pl.ds(g * LANES, LANES)]
            ddst = (g * LANES + lane) * DAY_DIM
            dsrc = dids * DAY_DIM
            for j in range(DAY_DIM):
                v = plsc.load_gather(dtab_v, [dsrc + j])
                plsc.store_scatter(dblk_v, [ddst + j], v)

        def pipe(ch, _):
            @pl.when(ch < n_chunks)
            def _():
                fire(ch)
                type_day_group(ch)

            @pl.when(ch > 0)
            def _():
                extract(ch - 1)

            return ()

        lax.fori_loop(0, n_chunks + 1, pipe, (), unroll=False)

        pltpu.sync_copy(
            dmblk_v, dm_out.at[pl.ds(base * DOMAIN_DIM, bpw * DOMAIN_DIM)])
        pltpu.sync_copy(
            tblk_v, t_out.at[pl.ds(base * TYPE_DIM, bpw * TYPE_DIM)])
        pltpu.sync_copy(
            dblk_v, d_out.at[pl.ds(base * DAY_DIM, bpw * DAY_DIM)])

    return pl.kernel(
        body,
        out_type=(
            jax.ShapeDtypeStruct((nb * DOMAIN_DIM,), jnp.float32),
            jax.ShapeDtypeStruct((nb * TYPE_DIM,), jnp.float32),
            jax.ShapeDtypeStruct((nb * DAY_DIM,), jnp.float32),
        ),
        mesh=plsc.VectorSubcoreMesh(core_axis_name="c",
                                    subcore_axis_name="s"),
        compiler_params=pltpu.CompilerParams(needs_layout_passes=False,
                                             use_tc_tiling_on_sc=True),
        scratch_types=(
            pltpu.VMEM((bpw,), jnp.int32),
            pltpu.VMEM((2 * DMA_CHUNK, DOMAIN_DIM, 128), jnp.float32),
            pltpu.VMEM((bpw,), jnp.int32),
            pltpu.VMEM((bpw,), jnp.int32),
            pltpu.VMEM((TYPE_VOCAB * TYPE_DIM,), jnp.float32),
            pltpu.VMEM((8 * DAY_DIM,), jnp.float32),
            pltpu.VMEM((bpw * DOMAIN_DIM,), jnp.float32),
            pltpu.VMEM((bpw * TYPE_DIM,), jnp.float32),
            pltpu.VMEM((bpw * DAY_DIM,), jnp.float32),
            pltpu.SemaphoreType.DMA,
        ),
    )


HB = B // 2
_sc_gather_half = _make_sc_gather(HB)

BLK = 2048


def _mlp_body(dm_ref, t_ref, d_ref, nf_ref,
              w1t_ref, w1d_ref, w1dm_ref, w1nf_ref,
              b1_ref, w2_ref, b2_ref, out_ref):
    h = jnp.dot(t_ref[...], w1t_ref[...], preferred_element_type=jnp.float32)
    h += jnp.dot(d_ref[...], w1d_ref[...], preferred_element_type=jnp.float32)
    h += jnp.dot(dm_ref[...], w1dm_ref[...], preferred_element_type=jnp.float32)
    h += jnp.dot(nf_ref[...], w1nf_ref[...], preferred_element_type=jnp.float32)
    h = jnp.maximum(h + b1_ref[...], 0.0)
    out = jnp.dot(h, w2_ref[...], preferred_element_type=jnp.float32)
    out_ref[...] = out + b2_ref[...]


def _mlp_half(dm, t, d, nf, w1t, w1d, w1dm, w1nf, b1r, W2, b2r):
    row_blk = lambda i: (i, 0)
    whole = lambda i: (0, 0)
    return pl.pallas_call(
        _mlp_body,
        grid=(HB // BLK,),
        in_specs=[
            pl.BlockSpec((BLK, DOMAIN_DIM), row_blk),
            pl.BlockSpec((BLK, TYPE_DIM), row_blk),
            pl.BlockSpec((BLK, DAY_DIM), row_blk),
            pl.BlockSpec((BLK, 4), row_blk),
            pl.BlockSpec((TYPE_DIM, HIDDEN), whole),
            pl.BlockSpec((DAY_DIM, HIDDEN), whole),
            pl.BlockSpec((DOMAIN_DIM, HIDDEN), whole),
            pl.BlockSpec((4, HIDDEN), whole),
            pl.BlockSpec((1, HIDDEN), whole),
            pl.BlockSpec((HIDDEN, 1), whole),
            pl.BlockSpec((1, 1), whole),
        ],
        out_specs=pl.BlockSpec((BLK, 1), row_blk),
        out_shape=jax.ShapeDtypeStruct((HB, 1), jnp.float32),
    )(dm, t, d, nf, w1t, w1d, w1dm, w1nf, b1r, W2, b2r)


def kernel(type_id, day_of_week_id, domain_id, hour_of_day, karma, descendants,
           type_table, day_table, domain_table, W1, b1, W2, b2):
    type_id = type_id.astype(jnp.int32)
    day_of_week_id = day_of_week_id.astype(jnp.int32)
    domain_id = domain_id.astype(jnp.int32)
    day_pad = jnp.pad(day_table, ((0, 8 - day_table.shape[0]), (0, 0)))
    dom_t = domain_table.T
    typ_flat = type_table.reshape(-1)
    day_flat = day_pad.reshape(-1)

    nf = jnp.stack([hour_of_day, karma, descendants,
                    jnp.zeros((B,), jnp.float32)], axis=1)
    w1t = W1[0:TYPE_DIM]
    w1d = W1[TYPE_DIM:TYPE_DIM + DAY_DIM]
    w1dm = W1[TYPE_DIM + DAY_DIM:TYPE_DIM + DAY_DIM + DOMAIN_DIM]
    w1nf = jnp.concatenate(
        [W1[28:31], jnp.zeros((1, HIDDEN), jnp.float32)], axis=0)
    b1r = b1[None, :]
    b2r = b2[None, :]

    outs = []
    for h in range(2):
        sl = slice(h * HB, (h + 1) * HB)
        dm, t, d = _sc_gather_half(dom_t, typ_flat, day_flat,
                                   domain_id[sl], type_id[sl],
                                   day_of_week_id[sl])
        outs.append(_mlp_half(
            dm.reshape(HB, DOMAIN_DIM), t.reshape(HB, TYPE_DIM),
            d.reshape(HB, DAY_DIM), nf[sl],
            w1t, w1d, w1dm, w1nf, b1r, W2, b2r))

    return jnp.concatenate(outs, axis=0)[:, 0]


# speedup vs baseline: 1.3429x; 1.3429x over previous
import functools

import jax
import jax.numpy as jnp
from jax import lax
from jax.experimental import pallas as pl
from jax.experimental.pallas import tpu as pltpu
from jax.experimental.pallas import tpu_sc as plsc

B = 16384
TYPE_VOCAB = 1000
DOMAIN_DIM = 16
TYPE_DIM = 8
DAY_DIM = 4
HIDDEN = 128
XW = 32

NC = 2
NS = 16
LANES = 16
NW = NC * NS
BPW = B // NW
GROUPS = BPW // LANES
DMA_CHUNK = 16
N_CHUNKS = BPW // DMA_CHUNK


def _sc_fuse_body(dom_t, typ_tab, day_tab, dom_id, typ_id, day_id,
                  hour, karma, desc, x_out,
                  didx_v, tiles_v, tid_v, did_v, hr_v, ka_v, de_v,
                  ttab_v, dtab_v, xblk_v, sem):
    wid = lax.axis_index("s") * NC + lax.axis_index("c")
    base = wid * BPW

    pltpu.sync_copy(dom_id.at[pl.ds(base, BPW)], didx_v)
    pltpu.sync_copy(typ_id.at[pl.ds(base, BPW)], tid_v)
    pltpu.sync_copy(day_id.at[pl.ds(base, BPW)], did_v)
    pltpu.sync_copy(hour.at[pl.ds(base, BPW)], hr_v)
    pltpu.sync_copy(karma.at[pl.ds(base, BPW)], ka_v)
    pltpu.sync_copy(desc.at[pl.ds(base, BPW)], de_v)
    pltpu.sync_copy(typ_tab, ttab_v)
    pltpu.sync_copy(day_tab, dtab_v)

    lane = lax.iota(jnp.int32, LANES)
    zeros16 = jnp.zeros((LANES,), jnp.float32)

    def fire(ch):
        ids = didx_v[pl.ds(ch * DMA_CHUNK, DMA_CHUNK)]
        ks = lax.shift_right_logical(ids, 7)
        slot_base = (ch % 2) * DMA_CHUNK
        for j in range(DMA_CHUNK):
            off = pl.multiple_of(ks[j] * 128, 128)
            pltpu.async_copy(dom_t.at[:, pl.ds(off, 128)],
                             tiles_v.at[slot_base + j], sem)

    def extract(ch):
        ids = didx_v[pl.ds(ch * DMA_CHUNK, DMA_CHUNK)]
        ls = ids & 127
        slot_base = (ch % 2) * DMA_CHUNK
        for j in range(DMA_CHUNK):
            pltpu.make_async_copy(dom_t.at[:, pl.ds(0, 128)],
                                  tiles_v.at[slot_base + j], sem).wait()
        for j in range(DMA_CHUNK):
            i = ch * DMA_CHUNK + j
            v = plsc.load_gather(
                tiles_v, [jnp.broadcast_to(slot_base + j, (LANES,)),
                          lane, jnp.broadcast_to(ls[j], (LANES,))])
            xblk_v[pl.ds(i * XW, DOMAIN_DIM)] = v

    def feature_group(g):
        rows = (g * LANES + lane) * XW

        tids = tid_v[pl.ds(g * LANES, LANES)]
        tsrc = tids * TYPE_DIM
        for j in range(TYPE_DIM):
            v = plsc.load_gather(ttab_v, [tsrc + j])
            plsc.store_scatter(xblk_v, [rows + (DOMAIN_DIM + j)], v)

        dids = did_v[pl.ds(g * LANES, LANES)]
        dsrc = dids * DAY_DIM
        for j in range(DAY_DIM):
            v = plsc.load_gather(dtab_v, [dsrc + j])
            plsc.store_scatter(xblk_v, [rows + (DOMAIN_DIM + TYPE_DIM + j)],
                               v)

        sl = pl.ds(g * LANES, LANES)
        plsc.store_scatter(xblk_v, [rows + 28], hr_v[sl])
        plsc.store_scatter(xblk_v, [rows + 29], ka_v[sl])
        plsc.store_scatter(xblk_v, [rows + 30], de_v[sl])
        plsc.store_scatter(xblk_v, [rows + 31], zeros16)

    def pipe(ch, _):
        @pl.when(ch < N_CHUNKS)
        def _():
            fire(ch)
            feature_group(ch)

        @pl.when(ch > 0)
        def _():
            extract(ch - 1)

        return ()

    lax.fori_loop(0, N_CHUNKS + 1, pipe, (), unroll=False)

    pltpu.sync_copy(xblk_v, x_out.at[pl.ds(base * XW, BPW * XW)])


_sc_fuse = functools.partial(
    pl.kernel,
    out_type=jax.ShapeDtypeStruct((B * XW,), jnp.float32),
    mesh=plsc.VectorSubcoreMesh(core_axis_name="c", subcore_axis_name="s"),
    compiler_params=pltpu.CompilerParams(needs_layout_passes=False,
                                         use_tc_tiling_on_sc=True),
    scratch_types=(
        pltpu.VMEM((BPW,), jnp.int32),
        pltpu.VMEM((2 * DMA_CHUNK, DOMAIN_DIM, 128), jnp.float32),
        pltpu.VMEM((BPW,), jnp.int32),
        pltpu.VMEM((BPW,), jnp.int32),
        pltpu.VMEM((BPW,), jnp.float32),
        pltpu.VMEM((BPW,), jnp.float32),
        pltpu.VMEM((BPW,), jnp.float32),
        pltpu.VMEM((TYPE_VOCAB * TYPE_DIM,), jnp.float32),
        pltpu.VMEM((8 * DAY_DIM,), jnp.float32),
        pltpu.VMEM((BPW * XW,), jnp.float32),
        pltpu.SemaphoreType.DMA,
    ),
)(_sc_fuse_body)


BLK = 2048


def _mlp_body(x_ref, w1_ref, b1_ref, w2t_ref, b2_ref, out_ref):
    h = jnp.dot(x_ref[...], w1_ref[...], preferred_element_type=jnp.float32)
    h = jnp.maximum(h + b1_ref[...], 0.0)
    out_ref[...] = jnp.sum(h * w2t_ref[...], axis=1) + b2_ref[0, 0]


def kernel(type_id, day_of_week_id, domain_id, hour_of_day, karma, descendants,
           type_table, day_table, domain_table, W1, b1, W2, b2):
    type_id = type_id.astype(jnp.int32)
    day_of_week_id = day_of_week_id.astype(jnp.int32)
    domain_id = domain_id.astype(jnp.int32)
    day_pad = jnp.pad(day_table, ((0, 8 - day_table.shape[0]), (0, 0)))

    x = _sc_fuse(domain_table.T, type_table.reshape(-1), day_pad.reshape(-1),
                 domain_id, type_id, day_of_week_id,
                 hour_of_day, karma, descendants)
    x = x.reshape(B, XW)

    w1p = jnp.concatenate(
        [W1[TYPE_DIM + DAY_DIM:TYPE_DIM + DAY_DIM + DOMAIN_DIM],
         W1[0:TYPE_DIM],
         W1[TYPE_DIM:TYPE_DIM + DAY_DIM],
         W1[28:31],
         jnp.zeros((1, HIDDEN), jnp.float32)], axis=0)

    row_blk = lambda i: (i, 0)
    whole = lambda i: (0, 0)
    out = pl.pallas_call(
        _mlp_body,
        grid=(B // BLK,),
        in_specs=[
            pl.BlockSpec((BLK, XW), row_blk),
            pl.BlockSpec((XW, HIDDEN), whole),
            pl.BlockSpec((1, HIDDEN), whole),
            pl.BlockSpec((1, HIDDEN), whole),
            pl.BlockSpec((1, 1), whole),
        ],
        out_specs=pl.BlockSpec((BLK,), lambda i: (i,)),
        out_shape=jax.ShapeDtypeStruct((B,), jnp.float32),
    )(x, w1p, b1[None, :], W2.T, b2[None, None, 0])
    return out


# speedup vs baseline: 1.3615x; 1.0138x over previous
import functools

import jax
import jax.numpy as jnp
from jax import lax
from jax.experimental import pallas as pl
from jax.experimental.pallas import tpu as pltpu
from jax.experimental.pallas import tpu_sc as plsc

B = 16384
TYPE_VOCAB = 1000
DOMAIN_DIM = 16
TYPE_DIM = 8
DAY_DIM = 4
HIDDEN = 128
XW = 32

NC = 2
NS = 16
LANES = 16
NW = NC * NS
BPW = B // NW
GROUPS = BPW // LANES
DMA_CHUNK = 16
N_CHUNKS = BPW // DMA_CHUNK


def _sc_fuse_body(dom_t, typ_tab, day_tab, dom_id, typ_id, day_id,
                  hour, karma, desc, x_out,
                  didx_v, tiles_v, tid_v, did_v, hr_v, ka_v, de_v,
                  ttab_v, dtab_v, xblk_v, sem):
    wid = lax.axis_index("s") * NC + lax.axis_index("c")
    base = wid * BPW

    pltpu.sync_copy(dom_id.at[pl.ds(base, BPW)], didx_v)
    pltpu.sync_copy(typ_id.at[pl.ds(base, BPW)], tid_v)
    pltpu.sync_copy(day_id.at[pl.ds(base, BPW)], did_v)
    pltpu.sync_copy(hour.at[pl.ds(base, BPW)], hr_v)
    pltpu.sync_copy(karma.at[pl.ds(base, BPW)], ka_v)
    pltpu.sync_copy(desc.at[pl.ds(base, BPW)], de_v)
    pltpu.sync_copy(typ_tab, ttab_v)
    pltpu.sync_copy(day_tab, dtab_v)

    lane = lax.iota(jnp.int32, LANES)
    zeros16 = jnp.zeros((LANES,), jnp.float32)

    def fire(ch):
        ids = didx_v[pl.ds(ch * DMA_CHUNK, DMA_CHUNK)]
        ks = lax.shift_right_logical(ids, 7)
        slot_base = (ch % 3) * DMA_CHUNK
        for j in range(DMA_CHUNK):
            off = pl.multiple_of(ks[j] * 128, 128)
            pltpu.async_copy(dom_t.at[:, pl.ds(off, 128)],
                             tiles_v.at[slot_base + j], sem)

    def extract(ch):
        ids = didx_v[pl.ds(ch * DMA_CHUNK, DMA_CHUNK)]
        ls = ids & 127
        slot_base = (ch % 3) * DMA_CHUNK
        for j in range(DMA_CHUNK):
            pltpu.make_async_copy(dom_t.at[:, pl.ds(0, 128)],
                                  tiles_v.at[slot_base + j], sem).wait()
        for j in range(DMA_CHUNK):
            i = ch * DMA_CHUNK + j
            v = plsc.load_gather(
                tiles_v, [jnp.broadcast_to(slot_base + j, (LANES,)),
                          lane, jnp.broadcast_to(ls[j], (LANES,))])
            xblk_v[pl.ds(i * XW, DOMAIN_DIM)] = v

    def feature_group(g):
        rows = (g * LANES + lane) * XW

        tids = tid_v[pl.ds(g * LANES, LANES)]
        tsrc = tids * TYPE_DIM
        for j in range(TYPE_DIM):
            v = plsc.load_gather(ttab_v, [tsrc + j])
            plsc.store_scatter(xblk_v, [rows + (DOMAIN_DIM + j)], v)

        dids = did_v[pl.ds(g * LANES, LANES)]
        dsrc = dids * DAY_DIM
        for j in range(DAY_DIM):
            v = plsc.load_gather(dtab_v, [dsrc + j])
            plsc.store_scatter(xblk_v, [rows + (DOMAIN_DIM + TYPE_DIM + j)],
                               v)

        sl = pl.ds(g * LANES, LANES)
        plsc.store_scatter(xblk_v, [rows + 28], hr_v[sl])
        plsc.store_scatter(xblk_v, [rows + 29], ka_v[sl])
        plsc.store_scatter(xblk_v, [rows + 30], de_v[sl])
        plsc.store_scatter(xblk_v, [rows + 31], zeros16)

    def pipe(ch, _):
        @pl.when(ch < N_CHUNKS)
        def _():
            fire(ch)
            feature_group(ch)

        @pl.when(ch > 1)
        def _():
            extract(ch - 2)

        return ()

    lax.fori_loop(0, N_CHUNKS + 2, pipe, (), unroll=False)

    pltpu.sync_copy(xblk_v, x_out.at[pl.ds(base * XW, BPW * XW)])


_sc_fuse = functools.partial(
    pl.kernel,
    out_type=jax.ShapeDtypeStruct((B * XW,), jnp.float32),
    mesh=plsc.VectorSubcoreMesh(core_axis_name="c", subcore_axis_name="s"),
    compiler_params=pltpu.CompilerParams(needs_layout_passes=False,
                                         use_tc_tiling_on_sc=True),
    scratch_types=(
        pltpu.VMEM((BPW,), jnp.int32),
        pltpu.VMEM((3 * DMA_CHUNK, DOMAIN_DIM, 128), jnp.float32),
        pltpu.VMEM((BPW,), jnp.int32),
        pltpu.VMEM((BPW,), jnp.int32),
        pltpu.VMEM((BPW,), jnp.float32),
        pltpu.VMEM((BPW,), jnp.float32),
        pltpu.VMEM((BPW,), jnp.float32),
        pltpu.VMEM((TYPE_VOCAB * TYPE_DIM,), jnp.float32),
        pltpu.VMEM((8 * DAY_DIM,), jnp.float32),
        pltpu.VMEM((BPW * XW,), jnp.float32),
        pltpu.SemaphoreType.DMA,
    ),
)(_sc_fuse_body)


BLK = 4096


def _mlp_body(x_ref, w1_ref, b1_ref, w2t_ref, b2_ref, out_ref):
    h = jnp.dot(x_ref[...], w1_ref[...], preferred_element_type=jnp.float32)
    h = jnp.maximum(h + b1_ref[...], 0.0)
    out_ref[...] = jnp.sum(h * w2t_ref[...], axis=1) + b2_ref[0, 0]


def kernel(type_id, day_of_week_id, domain_id, hour_of_day, karma, descendants,
           type_table, day_table, domain_table, W1, b1, W2, b2):
    type_id = type_id.astype(jnp.int32)
    day_of_week_id = day_of_week_id.astype(jnp.int32)
    domain_id = domain_id.astype(jnp.int32)
    day_pad = jnp.pad(day_table, ((0, 8 - day_table.shape[0]), (0, 0)))

    x = _sc_fuse(domain_table.T, type_table.reshape(-1), day_pad.reshape(-1),
                 domain_id, type_id, day_of_week_id,
                 hour_of_day, karma, descendants)
    x = x.reshape(B, XW)

    w1p = jnp.concatenate(
        [W1[TYPE_DIM + DAY_DIM:TYPE_DIM + DAY_DIM + DOMAIN_DIM],
         W1[0:TYPE_DIM],
         W1[TYPE_DIM:TYPE_DIM + DAY_DIM],
         W1[28:31],
         jnp.zeros((1, HIDDEN), jnp.float32)], axis=0)

    row_blk = lambda i: (i, 0)
    whole = lambda i: (0, 0)
    out = pl.pallas_call(
        _mlp_body,
        grid=(B // BLK,),
        in_specs=[
            pl.BlockSpec((BLK, XW), row_blk),
            pl.BlockSpec((XW, HIDDEN), whole),
            pl.BlockSpec((1, HIDDEN), whole),
            pl.BlockSpec((1, HIDDEN), whole),
            pl.BlockSpec((1, 1), whole),
        ],
        out_specs=pl.BlockSpec((BLK,), lambda i: (i,)),
        out_shape=jax.ShapeDtypeStruct((B,), jnp.float32),
    )(x, w1p, b1[None, :], W2.T, b2[None, None, 0])
    return out


# speedup vs baseline: 1.3629x; 1.0011x over previous
import functools

import jax
import jax.numpy as jnp
from jax import lax
from jax.experimental import pallas as pl
from jax.experimental.pallas import tpu as pltpu
from jax.experimental.pallas import tpu_sc as plsc

B = 16384
TYPE_VOCAB = 1000
DOMAIN_DIM = 16
TYPE_DIM = 8
DAY_DIM = 4
HIDDEN = 128
XW = 32

NC = 2
NS = 16
LANES = 16
NW = NC * NS
BPW = B // NW
GROUPS = BPW // LANES
DMA_CHUNK = 16
N_CHUNKS = BPW // DMA_CHUNK


def _sc_fuse_body(dom_t, typ_tab, day_tab, dom_id, typ_id, day_id,
                  hour, karma, desc, x_out,
                  didx_v, tiles_v, tid_v, did_v, hr_v, ka_v, de_v,
                  ttab_v, dtab_v, xblk_v, sem):
    wid = lax.axis_index("s") * NC + lax.axis_index("c")
    base = wid * BPW

    pltpu.sync_copy(dom_id.at[pl.ds(base, BPW)], didx_v)
    pltpu.sync_copy(typ_id.at[pl.ds(base, BPW)], tid_v)
    pltpu.sync_copy(day_id.at[pl.ds(base, BPW)], did_v)
    pltpu.sync_copy(hour.at[pl.ds(base, BPW)], hr_v)
    pltpu.sync_copy(karma.at[pl.ds(base, BPW)], ka_v)
    pltpu.sync_copy(desc.at[pl.ds(base, BPW)], de_v)
    pltpu.sync_copy(typ_tab, ttab_v)
    pltpu.sync_copy(day_tab, dtab_v)

    lane = lax.iota(jnp.int32, LANES)
    zeros16 = jnp.zeros((LANES,), jnp.float32)

    def fire(ch):
        ids = didx_v[pl.ds(ch * DMA_CHUNK, DMA_CHUNK)]
        ks = lax.shift_right_logical(ids, 7)
        slot_base = (ch % 3) * DMA_CHUNK
        for j in range(DMA_CHUNK):
            off = pl.multiple_of(ks[j] * 128, 128)
            pltpu.async_copy(dom_t.at[:, pl.ds(off, 128)],
                             tiles_v.at[slot_base + j], sem)

    def extract(ch):
        ids = didx_v[pl.ds(ch * DMA_CHUNK, DMA_CHUNK)]
        ls = ids & 127
        slot_base = (ch % 3) * DMA_CHUNK
        for j in range(DMA_CHUNK):
            pltpu.make_async_copy(dom_t.at[:, pl.ds(0, 128)],
                                  tiles_v.at[slot_base + j], sem).wait()
        for j in range(DMA_CHUNK):
            i = ch * DMA_CHUNK + j
            v = plsc.load_gather(
                tiles_v, [jnp.broadcast_to(slot_base + j, (LANES,)),
                          lane, jnp.broadcast_to(ls[j], (LANES,))])
            xblk_v[pl.ds(i * XW, DOMAIN_DIM)] = v

    def feature_group(g):
        rows = (g * LANES + lane) * XW

        tids = tid_v[pl.ds(g * LANES, LANES)]
        tsrc = tids * TYPE_DIM
        for j in range(TYPE_DIM):
            v = plsc.load_gather(ttab_v, [tsrc + j])
            plsc.store_scatter(xblk_v, [rows + (DOMAIN_DIM + j)], v)

        dids = did_v[pl.ds(g * LANES, LANES)]
        dsrc = dids * DAY_DIM
        for j in range(DAY_DIM):
            v = plsc.load_gather(dtab_v, [dsrc + j])
            plsc.store_scatter(xblk_v, [rows + (DOMAIN_DIM + TYPE_DIM + j)],
                               v)

        sl = pl.ds(g * LANES, LANES)
        plsc.store_scatter(xblk_v, [rows + 28], hr_v[sl])
        plsc.store_scatter(xblk_v, [rows + 29], ka_v[sl])
        plsc.store_scatter(xblk_v, [rows + 30], de_v[sl])
        plsc.store_scatter(xblk_v, [rows + 31], zeros16)

    def pipe(ch, _):
        @pl.when(ch < N_CHUNKS)
        def _():
            fire(ch)
            feature_group(ch)

        @pl.when(ch > 1)
        def _():
            extract(ch - 2)

        return ()

    lax.fori_loop(0, N_CHUNKS + 2, pipe, (), unroll=False)

    pltpu.sync_copy(xblk_v, x_out.at[pl.ds(base * XW, BPW * XW)])


_sc_fuse = functools.partial(
    pl.kernel,
    out_type=jax.ShapeDtypeStruct((B * XW,), jnp.float32),
    mesh=plsc.VectorSubcoreMesh(core_axis_name="c", subcore_axis_name="s"),
    compiler_params=pltpu.CompilerParams(needs_layout_passes=False,
                                         use_tc_tiling_on_sc=True),
    scratch_types=(
        pltpu.VMEM((BPW,), jnp.int32),
        pltpu.VMEM((3 * DMA_CHUNK, DOMAIN_DIM, 128), jnp.float32),
        pltpu.VMEM((BPW,), jnp.int32),
        pltpu.VMEM((BPW,), jnp.int32),
        pltpu.VMEM((BPW,), jnp.float32),
        pltpu.VMEM((BPW,), jnp.float32),
        pltpu.VMEM((BPW,), jnp.float32),
        pltpu.VMEM((TYPE_VOCAB * TYPE_DIM,), jnp.float32),
        pltpu.VMEM((8 * DAY_DIM,), jnp.float32),
        pltpu.VMEM((BPW * XW,), jnp.float32),
        pltpu.SemaphoreType.DMA,
    ),
)(_sc_fuse_body)


BLK = 8192


def _mlp_body(x_ref, w1_ref, b1_ref, w2t_ref, b2_ref, out_ref):
    h = jnp.dot(x_ref[...], w1_ref[...], preferred_element_type=jnp.float32)
    h = jnp.maximum(h + b1_ref[...], 0.0)
    out_ref[...] = jnp.sum(h * w2t_ref[...], axis=1) + b2_ref[0, 0]


def kernel(type_id, day_of_week_id, domain_id, hour_of_day, karma, descendants,
           type_table, day_table, domain_table, W1, b1, W2, b2):
    type_id = type_id.astype(jnp.int32)
    day_of_week_id = day_of_week_id.astype(jnp.int32)
    domain_id = domain_id.astype(jnp.int32)
    day_pad = jnp.pad(day_table, ((0, 8 - day_table.shape[0]), (0, 0)))

    x = _sc_fuse(domain_table.T, type_table.reshape(-1), day_pad.reshape(-1),
                 domain_id, type_id, day_of_week_id,
                 hour_of_day, karma, descendants)
    x = x.reshape(B, XW)

    w1p = jnp.concatenate(
        [W1[TYPE_DIM + DAY_DIM:TYPE_DIM + DAY_DIM + DOMAIN_DIM],
         W1[0:TYPE_DIM],
         W1[TYPE_DIM:TYPE_DIM + DAY_DIM],
         W1[28:31],
         jnp.zeros((1, HIDDEN), jnp.float32)], axis=0)

    row_blk = lambda i: (i, 0)
    whole = lambda i: (0, 0)
    out = pl.pallas_call(
        _mlp_body,
        grid=(B // BLK,),
        in_specs=[
            pl.BlockSpec((BLK, XW), row_blk),
            pl.BlockSpec((XW, HIDDEN), whole),
            pl.BlockSpec((1, HIDDEN), whole),
            pl.BlockSpec((1, HIDDEN), whole),
            pl.BlockSpec((1, 1), whole),
        ],
        out_specs=pl.BlockSpec((BLK,), lambda i: (i,)),
        out_shape=jax.ShapeDtypeStruct((B,), jnp.float32),
    )(x, w1p, b1[None, :], W2.T, b2[None, None, 0])
    return out


# speedup vs baseline: 1.3730x; 1.0074x over previous
import functools

import jax
import jax.numpy as jnp
from jax import lax
from jax.experimental import pallas as pl
from jax.experimental.pallas import tpu as pltpu
from jax.experimental.pallas import tpu_sc as plsc

B = 16384
TYPE_VOCAB = 1000
DOMAIN_DIM = 16
TYPE_DIM = 8
DAY_DIM = 4
HIDDEN = 128
XW = 32

NC = 2
NS = 16
LANES = 16
NW = NC * NS
BPW = B // NW
GROUPS = BPW // LANES
DMA_CHUNK = 16
N_CHUNKS = BPW // DMA_CHUNK


def _sc_fuse_body(dom_t, typ_tab, day_tab, dom_id, typ_id, day_id,
                  hour, karma, desc, x_out,
                  didx_v, tiles_v, tid_v, did_v, hr_v, ka_v, de_v,
                  ttab_v, dtab_v, xblk_v, sem):
    wid = lax.axis_index("s") * NC + lax.axis_index("c")
    base = wid * BPW

    pltpu.sync_copy(dom_id.at[pl.ds(base, BPW)], didx_v)
    pltpu.sync_copy(typ_id.at[pl.ds(base, BPW)], tid_v)
    pltpu.sync_copy(day_id.at[pl.ds(base, BPW)], did_v)
    pltpu.sync_copy(hour.at[pl.ds(base, BPW)], hr_v)
    pltpu.sync_copy(karma.at[pl.ds(base, BPW)], ka_v)
    pltpu.sync_copy(desc.at[pl.ds(base, BPW)], de_v)
    pltpu.sync_copy(typ_tab, ttab_v)
    pltpu.sync_copy(day_tab, dtab_v)

    lane = lax.iota(jnp.int32, LANES)
    zeros16 = jnp.zeros((LANES,), jnp.float32)

    def fire(ch):
        ids = didx_v[pl.ds(ch * DMA_CHUNK, DMA_CHUNK)]
        ks = lax.shift_right_logical(ids, 7)
        slot_base = (ch % 3) * DMA_CHUNK
        for j in range(DMA_CHUNK):
            off = pl.multiple_of(ks[j] * 128, 128)
            pltpu.async_copy(dom_t.at[:, pl.ds(off, 128)],
                             tiles_v.at[slot_base + j], sem)

    def extract(ch):
        ids = didx_v[pl.ds(ch * DMA_CHUNK, DMA_CHUNK)]
        ls = ids & 127
        slot_base = (ch % 3) * DMA_CHUNK
        for j in range(DMA_CHUNK):
            pltpu.make_async_copy(dom_t.at[:, pl.ds(0, 128)],
                                  tiles_v.at[slot_base + j], sem).wait()
        for j in range(DMA_CHUNK):
            i = ch * DMA_CHUNK + j
            v = plsc.load_gather(
                tiles_v, [jnp.broadcast_to(slot_base + j, (LANES,)),
                          lane, jnp.broadcast_to(ls[j], (LANES,))])
            xblk_v[pl.ds(i * XW, DOMAIN_DIM)] = v

    def feature_group(g):
        rows = (g * LANES + lane) * XW

        tids = tid_v[pl.ds(g * LANES, LANES)]
        tsrc = tids * TYPE_DIM
        for j in range(TYPE_DIM):
            v = plsc.load_gather(ttab_v, [tsrc + j])
            plsc.store_scatter(xblk_v, [rows + (DOMAIN_DIM + j)], v)

        dids = did_v[pl.ds(g * LANES, LANES)]
        dsrc = dids * DAY_DIM
        for j in range(DAY_DIM):
            v = plsc.load_gather(dtab_v, [dsrc + j])
            plsc.store_scatter(xblk_v, [rows + (DOMAIN_DIM + TYPE_DIM + j)],
                               v)

        sl = pl.ds(g * LANES, LANES)
        plsc.store_scatter(xblk_v, [rows + 28], hr_v[sl])
        plsc.store_scatter(xblk_v, [rows + 29], ka_v[sl])
        plsc.store_scatter(xblk_v, [rows + 30], de_v[sl])
        plsc.store_scatter(xblk_v, [rows + 31], zeros16)

    def pipe(ch, _):
        @pl.when(ch < N_CHUNKS)
        def _():
            fire(ch)
            feature_group(ch)

        @pl.when(ch > 1)
        def _():
            extract(ch - 2)

        return ()

    lax.fori_loop(0, N_CHUNKS + 2, pipe, (), unroll=False)

    pltpu.sync_copy(xblk_v, x_out.at[pl.ds(base * XW, BPW * XW)])


_sc_fuse = functools.partial(
    pl.kernel,
    out_type=jax.ShapeDtypeStruct((B * XW,), jnp.float32),
    mesh=plsc.VectorSubcoreMesh(core_axis_name="c", subcore_axis_name="s"),
    compiler_params=pltpu.CompilerParams(needs_layout_passes=False,
                                         use_tc_tiling_on_sc=True),
    scratch_types=(
        pltpu.VMEM((BPW,), jnp.int32),
        pltpu.VMEM((3 * DMA_CHUNK, DOMAIN_DIM, 128), jnp.float32),
        pltpu.VMEM((BPW,), jnp.int32),
        pltpu.VMEM((BPW,), jnp.int32),
        pltpu.VMEM((BPW,), jnp.float32),
        pltpu.VMEM((BPW,), jnp.float32),
        pltpu.VMEM((BPW,), jnp.float32),
        pltpu.VMEM((TYPE_VOCAB * TYPE_DIM,), jnp.float32),
        pltpu.VMEM((8 * DAY_DIM,), jnp.float32),
        pltpu.VMEM((BPW * XW,), jnp.float32),
        pltpu.SemaphoreType.DMA,
    ),
)(_sc_fuse_body)


BLK = 4096


def _mlp_body(x_ref, w1_ref, b1_ref, w2t_ref, b2_ref, out_ref):
    h = jnp.dot(x_ref[...], w1_ref[...], preferred_element_type=jnp.float32)
    h = jnp.maximum(h + b1_ref[...], 0.0)
    out_ref[...] = jnp.sum(h * w2t_ref[...], axis=1) + b2_ref[0, 0]


def kernel(type_id, day_of_week_id, domain_id, hour_of_day, karma, descendants,
           type_table, day_table, domain_table, W1, b1, W2, b2):
    type_id = type_id.astype(jnp.int32)
    day_of_week_id = day_of_week_id.astype(jnp.int32)
    domain_id = domain_id.astype(jnp.int32)
    day_pad = jnp.pad(day_table, ((0, 8 - day_table.shape[0]), (0, 0)))

    x = _sc_fuse(domain_table.T, type_table.reshape(-1), day_pad.reshape(-1),
                 domain_id, type_id, day_of_week_id,
                 hour_of_day, karma, descendants)
    x = x.reshape(B, XW)

    w1p = jnp.concatenate(
        [W1[TYPE_DIM + DAY_DIM:TYPE_DIM + DAY_DIM + DOMAIN_DIM],
         W1[0:TYPE_DIM],
         W1[TYPE_DIM:TYPE_DIM + DAY_DIM],
         W1[28:31],
         jnp.zeros((1, HIDDEN), jnp.float32)], axis=0)

    row_blk = lambda i: (i, 0)
    whole = lambda i: (0, 0)
    out = pl.pallas_call(
        _mlp_body,
        grid=(B // BLK,),
        in_specs=[
            pl.BlockSpec((BLK, XW), row_blk),
            pl.BlockSpec((XW, HIDDEN), whole),
            pl.BlockSpec((1, HIDDEN), whole),
            pl.BlockSpec((1, HIDDEN), whole),
            pl.BlockSpec((1, 1), whole),
        ],
        out_specs=pl.BlockSpec((BLK,), lambda i: (i,)),
        out_shape=jax.ShapeDtypeStruct((B,), jnp.float32),
    )(x, w1p, b1[None, :], W2.T, b2[None, None, 0])
    return out
